# baseline (speedup 1.0000x reference)
import functools

import jax
import jax.numpy as jnp
from jax.experimental import pallas as pl
from jax.experimental.pallas import tpu as pltpu

B, N, D = 64, 400, 128
EPS = 1e-5
GB = 8
GSTEPS = B // GB
K_TILES = 5
NT = N // K_TILES


def _dot_t(x, w):
    return jax.lax.dot_general(
        x.astype(jnp.bfloat16), w.astype(jnp.bfloat16),
        (((1,), (1,)), ((), ())), preferred_element_type=jnp.float32)


def _body(sc_ref, W1_ref, b1_ref, W2_ref, b2_ref, W3_ref, b3_ref,
          g1_ref, be1_ref, g2_ref, be2_ref, g3_ref, be3_ref,
          cW1_ref, cb1_ref, lg1_ref, lb1_ref,
          cW2_ref, cb2_ref, lg2_ref, lb2_ref, cW3_ref, cb3_ref,
          out_ref, xs_ref, acc_ref):
    i = pl.program_id(0)

    @pl.when(i < GSTEPS)
    def _gcn():
        sc = sc_ref[...]
        row = jax.lax.broadcasted_iota(jnp.int32, (GB, N, N), 1)
        col = jax.lax.broadcasted_iota(jnp.int32, (GB, N, N), 2)
        A = sc + jnp.where((row == col) & (sc == 0.0), 1.0, 0.0)
        deg_c = jnp.sum(A, axis=1, keepdims=True)
        dinv_c = jnp.where(deg_c > 0, jax.lax.rsqrt(deg_c), 0.0)
        dinv_r = jnp.transpose(dinv_c, (0, 2, 1))
        Ms = (A * dinv_r * dinv_c).astype(jnp.bfloat16)

        def prop(y):
            zs = [jax.lax.dot_general(Ms[g], y[g].astype(jnp.bfloat16),
                                      (((0,), (0,)), ((), ())),
                                      preferred_element_type=jnp.float32)
                  for g in range(GB)]
            return jnp.stack(zs, axis=0)

        def feat(x, w_ref):
            h = _dot_t(x.reshape(GB * N, x.shape[-1]), w_ref[...])
            return h.reshape(GB, N, D)

        def bn(x, g_ref, b_ref):
            mu = jnp.mean(x, axis=1, keepdims=True)
            var = jnp.mean((x - mu) ** 2, axis=1, keepdims=True)
            return (x - mu) * jax.lax.rsqrt(var + EPS) * g_ref[...] + b_ref[...]

        x = jax.nn.relu(bn(prop(feat(sc, W1_ref)) + b1_ref[...],
                           g1_ref, be1_ref))
        x = jax.nn.relu(bn(prop(feat(x, W2_ref)) + b2_ref[...],
                           g2_ref, be2_ref))
        x = bn(prop(feat(x, W3_ref)) + b3_ref[...], g3_ref, be3_ref)
        xs_ref[pl.ds(i * GB, GB)] = x.astype(jnp.bfloat16)

    @pl.when(i == GSTEPS)
    def _init():
        acc_ref[...] = jnp.zeros_like(acc_ref)

    @pl.when(i >= GSTEPS)
    def _clf():
        k = i - GSTEPS
        xt = xs_ref[:, pl.ds(k * NT, NT), :]
        xf = xt.reshape(B, NT * D)
        w = cW1_ref[...].astype(jnp.bfloat16)
        acc_ref[...] += jax.lax.dot_general(
            xf, w, (((1,), (1,)), ((), ())),
            preferred_element_type=jnp.float32)

    @pl.when(i == GSTEPS + K_TILES - 1)
    def _finish():
        def ln(x, g_ref, b_ref):
            mu = jnp.mean(x, axis=-1, keepdims=True)
            var = jnp.mean((x - mu) ** 2, axis=-1, keepdims=True)
            return (x - mu) * jax.lax.rsqrt(var + EPS) * g_ref[...] + b_ref[...]

        h = acc_ref[...] + cb1_ref[...]
        h = jax.nn.relu(ln(h, lg1_ref, lb1_ref))
        h = _dot_t(h, cW2_ref[...]) + cb2_ref[...]
        h = jax.nn.relu(ln(h, lg2_ref, lb2_ref))
        out_ref[...] = _dot_t(h, cW3_ref[...]) + cb3_ref[...]


def _full(spec_shape):
    nd = len(spec_shape)
    return pl.BlockSpec(spec_shape, lambda *_: (0,) * nd)


@functools.partial(jax.jit, static_argnames=("interpret",))
def kernel(fc_matrix, sc_matrix, W1, b1, W2, b2, W3, b3,
           g1, be1, g2, be2, g3, be3,
           cW1, cb1, lg1, lb1, cW2, cb2, lg2, lb2, cW3, cb3,
           interpret=False):
    del fc_matrix
    kt = NT * D
    logits = pl.pallas_call(
        _body,
        grid=(GSTEPS + K_TILES,),
        in_specs=[
            pl.BlockSpec((GB, N, N),
                         lambda i: (jnp.minimum(i, GSTEPS - 1), 0, 0)),
            _full((D, N)), _full((D,)),
            _full((D, D)), _full((D,)),
            _full((D, D)), _full((D,)),
            _full((D,)), _full((D,)),
            _full((D,)), _full((D,)),
            _full((D,)), _full((D,)),
            pl.BlockSpec((256, kt),
                         lambda i: (0, jnp.maximum(i - GSTEPS, 0))),
            _full((256,)), _full((256,)), _full((256,)),
            _full((64, 256)), _full((64,)), _full((64,)), _full((64,)),
            _full((2, 64)), _full((2,)),
        ],
        out_specs=pl.BlockSpec((B, 2), lambda i: (0, 0)),
        out_shape=jax.ShapeDtypeStruct((B, 2), jnp.float32),
        scratch_shapes=[pltpu.VMEM((B, N, D), jnp.bfloat16),
                        pltpu.VMEM((B, 256), jnp.float32)],
        compiler_params=pltpu.CompilerParams(
            dimension_semantics=("arbitrary",)),
        interpret=interpret,
    )(sc_matrix, W1, b1, W2, b2, W3, b3, g1, be1, g2, be2, g3, be3,
      cW1, cb1, lg1, lb1, cW2, cb2, lg2, lb2, cW3, cb3)
    return logits


# speedup vs baseline: 1.0016x; 1.0016x over previous
import jax
import jax.numpy as jnp
from jax.experimental import pallas as pl
from jax.experimental.pallas import tpu as pltpu

B, N, D = 64, 400, 128
EPS = 1e-5
GB = 8
GSTEPS = B // GB
K_TILES = 5
NT = N // K_TILES


def _dot_t(x, w):
    return jax.lax.dot_general(
        x.astype(jnp.bfloat16), w.astype(jnp.bfloat16),
        (((1,), (1,)), ((), ())), preferred_element_type=jnp.float32)


def _body(sc_ref, W1_ref, b1_ref, W2_ref, b2_ref, W3_ref, b3_ref,
          g1_ref, be1_ref, g2_ref, be2_ref, g3_ref, be3_ref,
          cW1_ref, cb1_ref, lg1_ref, lb1_ref,
          cW2_ref, cb2_ref, lg2_ref, lb2_ref, cW3_ref, cb3_ref,
          out_ref, xs_ref, acc_ref):
    i = pl.program_id(0)

    @pl.when(i < GSTEPS)
    def _gcn():
        sc = sc_ref[...]
        row = jax.lax.broadcasted_iota(jnp.int32, (GB, N, N), 1)
        col = jax.lax.broadcasted_iota(jnp.int32, (GB, N, N), 2)
        A = sc + jnp.where((row == col) & (sc == 0.0), 1.0, 0.0)
        deg_c = jnp.sum(A, axis=1, keepdims=True)
        dinv_c = jnp.where(deg_c > 0, jax.lax.rsqrt(deg_c), 0.0)
        dinv_r = jnp.transpose(dinv_c, (0, 2, 1))
        Ms = (A * dinv_r * dinv_c).astype(jnp.bfloat16)

        def prop(y):
            zs = [jax.lax.dot_general(Ms[g], y[g].astype(jnp.bfloat16),
                                      (((0,), (0,)), ((), ())),
                                      preferred_element_type=jnp.float32)
                  for g in range(GB)]
            return jnp.stack(zs, axis=0)

        def feat(x, w_ref):
            h = _dot_t(x.reshape(GB * N, x.shape[-1]), w_ref[...])
            return h.reshape(GB, N, D)

        def bn(x, g_ref, b_ref):
            mu = jnp.mean(x, axis=1, keepdims=True)
            var = jnp.mean((x - mu) ** 2, axis=1, keepdims=True)
            return (x - mu) * jax.lax.rsqrt(var + EPS) * g_ref[...] + b_ref[...]

        x = jax.nn.relu(bn(prop(feat(sc, W1_ref)) + b1_ref[...],
                           g1_ref, be1_ref))
        x = jax.nn.relu(bn(prop(feat(x, W2_ref)) + b2_ref[...],
                           g2_ref, be2_ref))
        x = bn(prop(feat(x, W3_ref)) + b3_ref[...], g3_ref, be3_ref)
        xs_ref[pl.ds(i * GB, GB)] = x.astype(jnp.bfloat16)

    @pl.when(i == GSTEPS)
    def _init():
        acc_ref[...] = jnp.zeros_like(acc_ref)

    @pl.when(i >= GSTEPS)
    def _clf():
        k = i - GSTEPS
        xt = xs_ref[:, pl.ds(k * NT, NT), :]
        xf = xt.reshape(B, NT * D)
        w = cW1_ref[...].astype(jnp.bfloat16)
        acc_ref[...] += jax.lax.dot_general(
            xf, w, (((1,), (1,)), ((), ())),
            preferred_element_type=jnp.float32)

    @pl.when(i == GSTEPS + K_TILES - 1)
    def _finish():
        def ln(x, g_ref, b_ref):
            mu = jnp.mean(x, axis=-1, keepdims=True)
            var = jnp.mean((x - mu) ** 2, axis=-1, keepdims=True)
            return (x - mu) * jax.lax.rsqrt(var + EPS) * g_ref[...] + b_ref[...]

        h = acc_ref[...] + cb1_ref[...]
        h = jax.nn.relu(ln(h, lg1_ref, lb1_ref))
        h = _dot_t(h, cW2_ref[...]) + cb2_ref[...]
        h = jax.nn.relu(ln(h, lg2_ref, lb2_ref))
        out_ref[...] = _dot_t(h, cW3_ref[...]) + cb3_ref[...]


def _full(spec_shape):
    nd = len(spec_shape)
    return pl.BlockSpec(spec_shape, lambda *_: (0,) * nd)


def kernel(fc_matrix, sc_matrix, W1, b1, W2, b2, W3, b3,
           g1, be1, g2, be2, g3, be3,
           cW1, cb1, lg1, lb1, cW2, cb2, lg2, lb2, cW3, cb3):
    del fc_matrix
    kt = NT * D
    logits = pl.pallas_call(
        _body,
        grid=(GSTEPS + K_TILES,),
        in_specs=[
            pl.BlockSpec((GB, N, N),
                         lambda i: (jnp.minimum(i, GSTEPS - 1), 0, 0)),
            _full((D, N)), _full((D,)),
            _full((D, D)), _full((D,)),
            _full((D, D)), _full((D,)),
            _full((D,)), _full((D,)),
            _full((D,)), _full((D,)),
            _full((D,)), _full((D,)),
            pl.BlockSpec((256, kt),
                         lambda i: (0, jnp.maximum(i - GSTEPS, 0))),
            _full((256,)), _full((256,)), _full((256,)),
            _full((64, 256)), _full((64,)), _full((64,)), _full((64,)),
            _full((2, 64)), _full((2,)),
        ],
        out_specs=pl.BlockSpec((B, 2), lambda i: (0, 0)),
        out_shape=jax.ShapeDtypeStruct((B, 2), jnp.float32),
        scratch_shapes=[pltpu.VMEM((B, N, D), jnp.bfloat16),
                        pltpu.VMEM((B, 256), jnp.float32)],
        compiler_params=pltpu.CompilerParams(
            dimension_semantics=("arbitrary",)),
    )(sc_matrix, W1, b1, W2, b2, W3, b3, g1, be1, g2, be2, g3, be3,
      cW1, cb1, lg1, lb1, cW2, cb2, lg2, lb2, cW3, cb3)
    return logits
